# baseline (speedup 1.0000x reference)
import functools

import jax
import jax.numpy as jnp
from jax import lax
from jax.experimental import pallas as pl
from jax.experimental.pallas import tpu as pltpu
from jax.experimental.pallas import tpu_sc as plsc

VOCAB = 100000
DIM = 64
BATCH = 1024
CTX = 20

_LANES = 16


def _make_pool_kernel(num_cores, num_subcores):
    nw = num_cores * num_subcores
    bpw = BATCH // nw
    ipw = bpw * CTX
    mesh = plsc.VectorSubcoreMesh(core_axis_name="c", subcore_axis_name="s")

    @functools.partial(
        pl.kernel,
        mesh=mesh,
        out_type=jax.ShapeDtypeStruct((BATCH, DIM), jnp.float32),
        scratch_types=[
            pltpu.VMEM((ipw,), jnp.int32),
            pltpu.VMEM((ipw, DIM), jnp.float32),
            pltpu.VMEM((bpw, DIM), jnp.float32),
            pltpu.SemaphoreType.DMA,
        ],
        compiler_params=pltpu.CompilerParams(use_tc_tiling_on_sc=False),
    )
    def pool(idx_hbm, table_hbm, out_hbm, idx_v, rows_v, pooled_v, sem):
        wid = lax.axis_index("s") * num_cores + lax.axis_index("c")
        pltpu.sync_copy(idx_hbm.at[pl.ds(wid * ipw, ipw)], idx_v)
        pltpu.async_copy(table_hbm.at[idx_v], rows_v, sem).wait()

        scale = jnp.float32(1.0 / CTX)

        def body(b, carry):
            row0 = b * CTX
            for c in range(DIM // _LANES):
                acc = rows_v[row0, pl.ds(c * _LANES, _LANES)]
                for j in range(1, CTX):
                    acc = acc + rows_v[row0 + j, pl.ds(c * _LANES, _LANES)]
                pooled_v[b, pl.ds(c * _LANES, _LANES)] = acc * scale
            return carry

        lax.fori_loop(0, bpw, body, 0)
        pltpu.sync_copy(pooled_v, out_hbm.at[pl.ds(wid * bpw, bpw)])

    return pool


def _matmul_body(pooled_ref, w_ref, b_ref, out_ref):
    acc = lax.dot_general(
        pooled_ref[...],
        w_ref[...],
        (((1,), (1,)), ((), ())),
        preferred_element_type=jnp.float32,
    )
    out_ref[...] = acc + b_ref[0, :][None, :]


_VB = 512


def _make_matmul():
    grid = pl.cdiv(VOCAB, _VB)
    return pl.pallas_call(
        _matmul_body,
        grid=(grid,),
        in_specs=[
            pl.BlockSpec((BATCH, DIM), lambda i: (0, 0)),
            pl.BlockSpec((_VB, DIM), lambda i: (i, 0)),
            pl.BlockSpec((1, _VB), lambda i: (0, i)),
        ],
        out_specs=pl.BlockSpec((BATCH, _VB), lambda i: (0, i)),
        out_shape=jax.ShapeDtypeStruct((BATCH, VOCAB), jnp.float32),
    )


def kernel(x, embed_weight, fc1_weight, fc1_bias):
    info = plsc.get_sparse_core_info()
    pool = _make_pool_kernel(info.num_cores, info.num_subcores)
    idx = x.reshape(-1).astype(jnp.int32)
    pooled = pool(idx, embed_weight)
    matmul = _make_matmul()
    return matmul(pooled, fc1_weight, fc1_bias.reshape(1, VOCAB))


# speedup vs baseline: 1.1295x; 1.1295x over previous
import functools

import jax
import jax.numpy as jnp
from jax import lax
from jax.experimental import pallas as pl
from jax.experimental.pallas import tpu as pltpu
from jax.experimental.pallas import tpu_sc as plsc

VOCAB = 100000
DIM = 64
BATCH = 1024
CTX = 20

_LANES = 16


def _make_pool_kernel(num_cores, num_subcores):
    nw = num_cores * num_subcores
    bpw = BATCH // nw
    ipw = bpw * CTX
    mesh = plsc.VectorSubcoreMesh(core_axis_name="c", subcore_axis_name="s")

    @functools.partial(
        pl.kernel,
        mesh=mesh,
        out_type=jax.ShapeDtypeStruct((BATCH, DIM), jnp.float32),
        scratch_types=[
            pltpu.VMEM((ipw,), jnp.int32),
            pltpu.VMEM((ipw, DIM), jnp.float32),
            pltpu.VMEM((bpw, DIM), jnp.float32),
            pltpu.SemaphoreType.DMA,
        ],
        compiler_params=pltpu.CompilerParams(use_tc_tiling_on_sc=False),
    )
    def pool(idx_hbm, table_hbm, out_hbm, idx_v, rows_v, pooled_v, sem):
        wid = lax.axis_index("s") * num_cores + lax.axis_index("c")
        pltpu.sync_copy(idx_hbm.at[pl.ds(wid * ipw, ipw)], idx_v)
        pltpu.async_copy(table_hbm.at[idx_v], rows_v, sem).wait()

        scale = jnp.float32(1.0 / CTX)

        def body(b, carry):
            row0 = b * CTX
            for c in range(DIM // _LANES):
                acc = rows_v[row0, pl.ds(c * _LANES, _LANES)]
                for j in range(1, CTX):
                    acc = acc + rows_v[row0 + j, pl.ds(c * _LANES, _LANES)]
                pooled_v[b, pl.ds(c * _LANES, _LANES)] = acc * scale
            return carry

        lax.fori_loop(0, bpw, body, 0)
        pltpu.sync_copy(pooled_v, out_hbm.at[pl.ds(wid * bpw, bpw)])

    return pool


def _matmul_body(pooled_ref, w_ref, b_ref, out_ref):
    acc = lax.dot_general(
        pooled_ref[...],
        w_ref[...],
        (((1,), (1,)), ((), ())),
        preferred_element_type=jnp.float32,
    )
    out_ref[...] = acc + b_ref[0, :][None, :]


_VB = 2048


def _make_matmul():
    grid = pl.cdiv(VOCAB, _VB)
    return pl.pallas_call(
        _matmul_body,
        grid=(grid,),
        in_specs=[
            pl.BlockSpec((BATCH, DIM), lambda i: (0, 0)),
            pl.BlockSpec((_VB, DIM), lambda i: (i, 0)),
            pl.BlockSpec((1, _VB), lambda i: (0, i)),
        ],
        out_specs=pl.BlockSpec((BATCH, _VB), lambda i: (0, i)),
        out_shape=jax.ShapeDtypeStruct((BATCH, VOCAB), jnp.float32),
    )


def kernel(x, embed_weight, fc1_weight, fc1_bias):
    info = plsc.get_sparse_core_info()
    pool = _make_pool_kernel(info.num_cores, info.num_subcores)
    idx = x.reshape(-1).astype(jnp.int32)
    pooled = pool(idx, embed_weight)
    matmul = _make_matmul()
    return matmul(pooled, fc1_weight, fc1_bias.reshape(1, VOCAB))
